# baseline (speedup 1.0000x reference)
import functools

import jax
import jax.numpy as jnp
from jax import lax
from jax.experimental import pallas as pl
from jax.experimental.pallas import tpu as pltpu
from jax.experimental.pallas import tpu_sc as plsc

LANES = 16
NUM_CORES = 2
NUM_SUBCORES = 16
NW = NUM_CORES * NUM_SUBCORES
CHUNK = 64


def _sc_body(S, D, nchunks, ids_hbm, pos_hbm, tok_hbm, out_hbm,
             idx_v, pos_v, buf, gsem):
    cid = lax.axis_index("c")
    sid = lax.axis_index("s")
    wid = sid * NUM_CORES + cid
    rows_per_w = nchunks * CHUNK
    base = wid * rows_per_w

    pltpu.sync_copy(ids_hbm.at[pl.ds(wid * nchunks, nchunks)], idx_v)
    pltpu.sync_copy(pos_hbm, pos_v)

    def chunk_step(c, carry):
        pltpu.async_copy(tok_hbm.at[idx_v.at[c]], buf, gsem).wait()

        def row_step(r, carry2):
            p = lax.rem(r, S)
            for j in range(D // LANES):
                sl = pl.ds(j * LANES, LANES)
                buf[r, sl] = buf[r, sl] + pos_v[p, sl]
            return carry2

        lax.fori_loop(0, CHUNK, row_step, 0, unroll=False)

        pltpu.sync_copy(buf, out_hbm.at[pl.ds(base + c * CHUNK, CHUNK)])
        return carry

    lax.fori_loop(0, nchunks, chunk_step, 0, unroll=False)


def kernel(input_ids, token_embedding, position_embedding):
    B, S = input_ids.shape
    V, D = token_embedding.shape
    n_flat = B * S
    assert n_flat % (NW * CHUNK) == 0 and CHUNK % S == 0 and D % LANES == 0
    nchunks = n_flat // (NW * CHUNK)

    ids2d = input_ids.reshape(NW * nchunks, CHUNK).astype(jnp.int32)

    mesh = plsc.VectorSubcoreMesh(core_axis_name="c", subcore_axis_name="s")
    body = functools.partial(_sc_body, S, D, nchunks)
    out = pl.kernel(
        body,
        out_type=jax.ShapeDtypeStruct((n_flat, D), jnp.float32),
        mesh=mesh,
        scratch_types=[
            pltpu.VMEM((nchunks, CHUNK), jnp.int32),
            pltpu.VMEM((S, D), jnp.float32),
            pltpu.VMEM((CHUNK, D), jnp.float32),
            pltpu.SemaphoreType.DMA,
        ],
    )(ids2d, position_embedding, token_embedding)
    return out.reshape(B, S, D)


# speedup vs baseline: 1.2003x; 1.2003x over previous
import functools

import jax
import jax.numpy as jnp
from jax import lax
from jax.experimental import pallas as pl
from jax.experimental.pallas import tpu as pltpu
from jax.experimental.pallas import tpu_sc as plsc

LANES = 16
NUM_CORES = 2
NUM_SUBCORES = 16
NW = NUM_CORES * NUM_SUBCORES
CHUNK = 64


def _sc_body(S, D, nchunks, ids_hbm, pos_hbm, tok_hbm, out_hbm,
             idx_v, pos_v, buf0, buf1, gsem0, gsem1):
    cid = lax.axis_index("c")
    sid = lax.axis_index("s")
    wid = sid * NUM_CORES + cid
    rows_per_w = nchunks * CHUNK
    base = wid * rows_per_w

    pltpu.sync_copy(ids_hbm.at[pl.ds(wid * nchunks, nchunks)], idx_v)
    pltpu.sync_copy(pos_hbm, pos_v)

    bufs = (buf0, buf1)
    gsems = (gsem0, gsem1)

    def add_pos(buf):
        def row_step(r, carry2):
            p = lax.rem(r, S)
            for j in range(D // LANES):
                sl = pl.ds(j * LANES, LANES)
                buf[r, sl] = buf[r, sl] + pos_v[p, sl]
            return carry2

        lax.fori_loop(0, CHUNK, row_step, 0, unroll=False)

    pltpu.async_copy(tok_hbm.at[idx_v.at[0]], buf0, gsem0)
    pltpu.async_copy(tok_hbm.at[idx_v.at[1]], buf1, gsem1)

    def pair_step(i, carry):
        c0 = i * 2
        for b in range(2):
            c = c0 + b
            buf, gsem = bufs[b], gsems[b]
            pltpu.make_async_copy(tok_hbm.at[idx_v.at[c]], buf, gsem).wait()
            add_pos(buf)
            pltpu.sync_copy(buf, out_hbm.at[pl.ds(base + c * CHUNK, CHUNK)])
            nc = c + 2

            @pl.when(nc < nchunks)
            def _():
                pltpu.async_copy(tok_hbm.at[idx_v.at[nc]], buf, gsem)

        return carry

    lax.fori_loop(0, nchunks // 2, pair_step, 0, unroll=False)


def kernel(input_ids, token_embedding, position_embedding):
    B, S = input_ids.shape
    V, D = token_embedding.shape
    n_flat = B * S
    assert n_flat % (NW * CHUNK) == 0 and CHUNK % S == 0 and D % LANES == 0
    nchunks = n_flat // (NW * CHUNK)

    ids2d = input_ids.reshape(NW * nchunks, CHUNK).astype(jnp.int32)

    mesh = plsc.VectorSubcoreMesh(core_axis_name="c", subcore_axis_name="s")
    body = functools.partial(_sc_body, S, D, nchunks)
    out = pl.kernel(
        body,
        out_type=jax.ShapeDtypeStruct((n_flat, D), jnp.float32),
        mesh=mesh,
        scratch_types=[
            pltpu.VMEM((nchunks, CHUNK), jnp.int32),
            pltpu.VMEM((S, D), jnp.float32),
            pltpu.VMEM((CHUNK, D), jnp.float32),
            pltpu.VMEM((CHUNK, D), jnp.float32),
            pltpu.SemaphoreType.DMA,
            pltpu.SemaphoreType.DMA,
        ],
    )(ids2d, position_embedding, token_embedding)
    return out.reshape(B, S, D)
